# Initial kernel scaffold; baseline (speedup 1.0000x reference)
import jax
import jax.numpy as jnp
from jax.experimental import pallas as pl


def kernel(x, lengths, table, W1, b1, W2, b2):
    raise NotImplementedError("write your pallas kernel here")


# speedup vs baseline: 7.6641x; 7.6641x over previous
import functools

import jax
import jax.numpy as jnp
from jax import lax
from jax.experimental import pallas as pl
from jax.experimental.pallas import tpu as pltpu
from jax.experimental.pallas import tpu_sc as plsc

B = 16384
HIST = 200
D = 16
HIDDEN = 100
OUT = 3

NC = 2
NS = 16
NW = NC * NS
RPW = B // NW
CHUNK = 8
S = 100
NSTREAM = CHUNK * HIST // S
NCHUNK = RPW // CHUNK


def _pool_body(x_hbm, table_hbm, rep_hbm, idx_v, rows_v, out_v, sem):
    wid = lax.axis_index("s") * NC + lax.axis_index("c")

    def chunk_body(ci, carry):
        row0 = wid * RPW + ci * CHUNK
        pltpu.sync_copy(x_hbm.at[pl.ds(row0 * 2, NSTREAM)], idx_v)
        cps = [
            pltpu.async_copy(table_hbm.at[idx_v.at[j]], rows_v.at[j], sem)
            for j in range(NSTREAM)
        ]
        for c in cps:
            c.wait()
        for r in range(CHUNK):
            def body(j, accs):
                a0, a1 = accs
                a0 = a0 + rows_v[2 * r, j, :]
                a1 = a1 + rows_v[2 * r + 1, j, :]
                return (a0, a1)
            z = jnp.zeros((16,), jnp.float32)
            a0, a1 = lax.fori_loop(0, S, body, (z, z), unroll=4)
            out_v[r, :] = a0 + a1
        pltpu.sync_copy(out_v, rep_hbm.at[pl.ds(row0, CHUNK)])
        return carry

    lax.fori_loop(0, NCHUNK, chunk_body, 0)


def _pooled_sum(x, table):
    x2 = x.reshape(2 * B, S).astype(jnp.int32)
    mesh = plsc.VectorSubcoreMesh(core_axis_name="c", subcore_axis_name="s")
    f = functools.partial(
        pl.kernel,
        mesh=mesh,
        out_type=jax.ShapeDtypeStruct((B, D), jnp.float32),
        scratch_types=[
            pltpu.VMEM((NSTREAM, S), jnp.int32),
            pltpu.VMEM((NSTREAM, S, D), jnp.float32),
            pltpu.VMEM((CHUNK, D), jnp.float32),
            pltpu.SemaphoreType.DMA,
        ],
        compiler_params=pltpu.CompilerParams(use_tc_tiling_on_sc=False),
    )(_pool_body)
    return f(x2, table)


def _mlp_body(rep_ref, len_ref, w1_ref, b1_ref, w2_ref, b2_ref, out_ref):
    rep = rep_ref[...] / len_ref[...]
    h = jnp.dot(rep, w1_ref[...], preferred_element_type=jnp.float32)
    h = jnp.maximum(h + b1_ref[...], 0.0)
    o = jnp.dot(h, w2_ref[...], preferred_element_type=jnp.float32)
    out_ref[...] = o + b2_ref[...]


def _mlp(rep_sum, lenf, W1, b1, W2, b2):
    H_PAD = 128
    O_PAD = 128
    W1p = jnp.zeros((D, H_PAD), jnp.float32).at[:, :HIDDEN].set(W1)
    b1p = jnp.zeros((1, H_PAD), jnp.float32).at[:, :HIDDEN].set(b1)
    W2p = jnp.zeros((H_PAD, O_PAD), jnp.float32).at[:HIDDEN, :OUT].set(W2)
    b2p = jnp.zeros((1, O_PAD), jnp.float32).at[:, :OUT].set(b2)
    BLK = 2048
    grid = (B // BLK,)
    out = pl.pallas_call(
        _mlp_body,
        grid=grid,
        in_specs=[
            pl.BlockSpec((BLK, D), lambda i: (i, 0)),
            pl.BlockSpec((BLK, 1), lambda i: (i, 0)),
            pl.BlockSpec((D, H_PAD), lambda i: (0, 0)),
            pl.BlockSpec((1, H_PAD), lambda i: (0, 0)),
            pl.BlockSpec((H_PAD, O_PAD), lambda i: (0, 0)),
            pl.BlockSpec((1, O_PAD), lambda i: (0, 0)),
        ],
        out_specs=pl.BlockSpec((BLK, O_PAD), lambda i: (i, 0)),
        out_shape=jax.ShapeDtypeStruct((B, O_PAD), jnp.float32),
    )(rep_sum, lenf, W1p, b1p, W2p, b2p)
    return out[:, :OUT]


def kernel(x, lengths, table, W1, b1, W2, b2):
    rep_sum = _pooled_sum(x, table)
    lenf = lengths.astype(jnp.float32).reshape(B, 1)
    return _mlp(rep_sum, lenf, W1, b1, W2, b2)


# speedup vs baseline: 9.3594x; 1.2212x over previous
import functools

import jax
import jax.numpy as jnp
from jax import lax
from jax.experimental import pallas as pl
from jax.experimental.pallas import tpu as pltpu
from jax.experimental.pallas import tpu_sc as plsc

B = 16384
HIST = 200
D = 16
HIDDEN = 100
OUT = 3

NC = 2
NS = 16
NW = NC * NS
RPW = B // NW
CHUNK = 16
S = 128
IDX_PER_CHUNK = CHUNK * HIST
NSTREAM = IDX_PER_CHUNK // S
NCHUNK = RPW // CHUNK
XROWS_PER_CHUNK = NSTREAM


def _fire(x_hbm, table_hbm, idx_v, rows_v, sem, wid, ci, b):
    xrow0 = wid * (NCHUNK * XROWS_PER_CHUNK) + ci * XROWS_PER_CHUNK
    pltpu.sync_copy(x_hbm.at[pl.ds(xrow0, XROWS_PER_CHUNK)], idx_v.at[b])
    for j in range(NSTREAM):
        pltpu.async_copy(
            table_hbm.at[idx_v.at[b].at[j]],
            rows_v.at[b].at[pl.ds(j * S, S)],
            sem,
        )


def _drain_gather(table_hbm, rows_v, sem, b):
    pltpu.make_async_copy(
        table_hbm.at[pl.ds(0, IDX_PER_CHUNK)], rows_v.at[b], sem
    ).wait()


def _compute(rows_v, out_v, rep_hbm, out_sem, wid, ci, b, drain_prev):
    @pl.when(drain_prev)
    def _():
        pltpu.make_async_copy(
            out_v.at[b], rep_hbm.at[pl.ds(0, CHUNK)], out_sem
        ).wait()

    for r in range(CHUNK):
        base = r * HIST
        z = jnp.zeros((16,), jnp.float32)

        def body(j, accs):
            a0, a1, a2, a3 = accs
            a0 = a0 + rows_v[b, base + j, :]
            a1 = a1 + rows_v[b, base + 50 + j, :]
            a2 = a2 + rows_v[b, base + 100 + j, :]
            a3 = a3 + rows_v[b, base + 150 + j, :]
            return (a0, a1, a2, a3)

        a0, a1, a2, a3 = lax.fori_loop(0, 50, body, (z, z, z, z), unroll=2)
        out_v[b, r, :] = (a0 + a1) + (a2 + a3)
    row0 = wid * RPW + ci * CHUNK
    pltpu.async_copy(out_v.at[b], rep_hbm.at[pl.ds(row0, CHUNK)], out_sem)


def _pool_body(x_hbm, table_hbm, rep_hbm, idx_v, rows_v, out_v, sem0, sem1,
               out_sem):
    wid = lax.axis_index("s") * NC + lax.axis_index("c")

    _fire(x_hbm, table_hbm, idx_v, rows_v, sem0, wid, 0, 0)

    def pair_body(i, carry):
        c0 = 2 * i
        _fire(x_hbm, table_hbm, idx_v, rows_v, sem1, wid, c0 + 1, 1)
        _drain_gather(table_hbm, rows_v, sem0, 0)
        _compute(rows_v, out_v, rep_hbm, out_sem, wid, c0, 0, i > 0)

        @pl.when(i < NCHUNK // 2 - 1)
        def _():
            _fire(x_hbm, table_hbm, idx_v, rows_v, sem0, wid, c0 + 2, 0)

        _drain_gather(table_hbm, rows_v, sem1, 1)
        _compute(rows_v, out_v, rep_hbm, out_sem, wid, c0 + 1, 1, i > 0)
        return carry

    lax.fori_loop(0, NCHUNK // 2, pair_body, 0)

    pltpu.make_async_copy(out_v.at[0], rep_hbm.at[pl.ds(0, CHUNK)],
                          out_sem).wait()
    pltpu.make_async_copy(out_v.at[1], rep_hbm.at[pl.ds(0, CHUNK)],
                          out_sem).wait()


def _pooled_sum(x, table):
    x2 = x.reshape(B * HIST // S, S).astype(jnp.int32)
    mesh = plsc.VectorSubcoreMesh(core_axis_name="c", subcore_axis_name="s")
    f = functools.partial(
        pl.kernel,
        mesh=mesh,
        out_type=jax.ShapeDtypeStruct((B, D), jnp.float32),
        scratch_types=[
            pltpu.VMEM((2, NSTREAM, S), jnp.int32),
            pltpu.VMEM((2, IDX_PER_CHUNK, D), jnp.float32),
            pltpu.VMEM((2, CHUNK, D), jnp.float32),
            pltpu.SemaphoreType.DMA,
            pltpu.SemaphoreType.DMA,
            pltpu.SemaphoreType.DMA,
        ],
        compiler_params=pltpu.CompilerParams(use_tc_tiling_on_sc=False),
    )(_pool_body)
    return f(x2, table)


def _mlp_body(rep_ref, len_ref, w1_ref, b1_ref, w2_ref, b2_ref, out_ref):
    rep = rep_ref[...] / len_ref[...]
    h = jnp.dot(rep, w1_ref[...], preferred_element_type=jnp.float32)
    h = jnp.maximum(h + b1_ref[...], 0.0)
    o = jnp.dot(h, w2_ref[...], preferred_element_type=jnp.float32)
    out_ref[...] = o + b2_ref[...]


def _mlp(rep_sum, lenf, W1, b1, W2, b2):
    H_PAD = 128
    O_PAD = 128
    W1p = jnp.zeros((D, H_PAD), jnp.float32).at[:, :HIDDEN].set(W1)
    b1p = jnp.zeros((1, H_PAD), jnp.float32).at[:, :HIDDEN].set(b1)
    W2p = jnp.zeros((H_PAD, O_PAD), jnp.float32).at[:HIDDEN, :OUT].set(W2)
    b2p = jnp.zeros((1, O_PAD), jnp.float32).at[:, :OUT].set(b2)
    BLK = 2048
    grid = (B // BLK,)
    out = pl.pallas_call(
        _mlp_body,
        grid=grid,
        in_specs=[
            pl.BlockSpec((BLK, D), lambda i: (i, 0)),
            pl.BlockSpec((BLK, 1), lambda i: (i, 0)),
            pl.BlockSpec((D, H_PAD), lambda i: (0, 0)),
            pl.BlockSpec((1, H_PAD), lambda i: (0, 0)),
            pl.BlockSpec((H_PAD, O_PAD), lambda i: (0, 0)),
            pl.BlockSpec((1, O_PAD), lambda i: (0, 0)),
        ],
        out_specs=pl.BlockSpec((BLK, O_PAD), lambda i: (i, 0)),
        out_shape=jax.ShapeDtypeStruct((B, O_PAD), jnp.float32),
    )(rep_sum, lenf, W1p, b1p, W2p, b2p)
    return out[:, :OUT]


def kernel(x, lengths, table, W1, b1, W2, b2):
    rep_sum = _pooled_sum(x, table)
    lenf = lengths.astype(jnp.float32).reshape(B, 1)
    return _mlp(rep_sum, lenf, W1, b1, W2, b2)


# speedup vs baseline: 9.3660x; 1.0007x over previous
import functools

import jax
import jax.numpy as jnp
from jax import lax
from jax.experimental import pallas as pl
from jax.experimental.pallas import tpu as pltpu
from jax.experimental.pallas import tpu_sc as plsc

B = 16384
HIST = 200
D = 16
HIDDEN = 100
OUT = 3

NC = 2
NS = 16
NW = NC * NS
RPW = B // NW
CHUNK = 16
S0 = 104
S1 = HIST - S0
IDX_PER_CHUNK = CHUNK * HIST
NCHUNK = RPW // CHUNK


def _fire(x_hbm, table_hbm, idx_v, rows_v, sem, wid, ci, b):
    row0 = wid * RPW + ci * CHUNK
    pltpu.sync_copy(x_hbm.at[pl.ds(row0, CHUNK)], idx_v.at[b])
    for r in range(CHUNK):
        pltpu.async_copy(
            table_hbm.at[idx_v.at[b].at[r].at[pl.ds(0, S0)]],
            rows_v.at[b].at[pl.ds(r * HIST, S0)],
            sem,
        )
        pltpu.async_copy(
            table_hbm.at[idx_v.at[b].at[r].at[pl.ds(S0, S1)]],
            rows_v.at[b].at[pl.ds(r * HIST + S0, S1)],
            sem,
        )


def _drain_gather(table_hbm, rows_v, sem, b):
    pltpu.make_async_copy(
        table_hbm.at[pl.ds(0, IDX_PER_CHUNK)], rows_v.at[b], sem
    ).wait()


def _compute(rows_v, out_v, rep_hbm, out_sem, wid, ci, b, drain_prev):
    @pl.when(drain_prev)
    def _():
        pltpu.make_async_copy(
            out_v.at[b], rep_hbm.at[pl.ds(0, CHUNK)], out_sem
        ).wait()

    for r in range(CHUNK):
        base = r * HIST
        z = jnp.zeros((16,), jnp.float32)

        def body(j, accs):
            a0, a1, a2, a3 = accs
            a0 = a0 + rows_v[b, base + j, :]
            a1 = a1 + rows_v[b, base + 50 + j, :]
            a2 = a2 + rows_v[b, base + 100 + j, :]
            a3 = a3 + rows_v[b, base + 150 + j, :]
            return (a0, a1, a2, a3)

        a0, a1, a2, a3 = lax.fori_loop(0, 50, body, (z, z, z, z), unroll=2)
        out_v[b, r, :] = (a0 + a1) + (a2 + a3)
    row0 = wid * RPW + ci * CHUNK
    pltpu.async_copy(out_v.at[b], rep_hbm.at[pl.ds(row0, CHUNK)], out_sem)


def _pool_body(x_hbm, table_hbm, rep_hbm, idx_v, rows_v, out_v, sem0, sem1,
               out_sem):
    wid = lax.axis_index("s") * NC + lax.axis_index("c")

    _fire(x_hbm, table_hbm, idx_v, rows_v, sem0, wid, 0, 0)

    def pair_body(i, carry):
        c0 = 2 * i
        _fire(x_hbm, table_hbm, idx_v, rows_v, sem1, wid, c0 + 1, 1)
        _drain_gather(table_hbm, rows_v, sem0, 0)
        _compute(rows_v, out_v, rep_hbm, out_sem, wid, c0, 0, i > 0)

        @pl.when(i < NCHUNK // 2 - 1)
        def _():
            _fire(x_hbm, table_hbm, idx_v, rows_v, sem0, wid, c0 + 2, 0)

        _drain_gather(table_hbm, rows_v, sem1, 1)
        _compute(rows_v, out_v, rep_hbm, out_sem, wid, c0 + 1, 1, i > 0)
        return carry

    lax.fori_loop(0, NCHUNK // 2, pair_body, 0)

    pltpu.make_async_copy(out_v.at[0], rep_hbm.at[pl.ds(0, CHUNK)],
                          out_sem).wait()
    pltpu.make_async_copy(out_v.at[1], rep_hbm.at[pl.ds(0, CHUNK)],
                          out_sem).wait()


def _pooled_sum(x, table):
    x2 = x.astype(jnp.int32)
    mesh = plsc.VectorSubcoreMesh(core_axis_name="c", subcore_axis_name="s")
    f = functools.partial(
        pl.kernel,
        mesh=mesh,
        out_type=jax.ShapeDtypeStruct((B, D), jnp.float32),
        scratch_types=[
            pltpu.VMEM((2, CHUNK, HIST), jnp.int32),
            pltpu.VMEM((2, IDX_PER_CHUNK, D), jnp.float32),
            pltpu.VMEM((2, CHUNK, D), jnp.float32),
            pltpu.SemaphoreType.DMA,
            pltpu.SemaphoreType.DMA,
            pltpu.SemaphoreType.DMA,
        ],
        compiler_params=pltpu.CompilerParams(use_tc_tiling_on_sc=False),
    )(_pool_body)
    return f(x2, table)


def _mlp_body(rep_ref, len_ref, w1_ref, b1_ref, w2_ref, b2_ref, out_ref):
    rep = rep_ref[...] / len_ref[...]
    h = jnp.dot(rep, w1_ref[...], preferred_element_type=jnp.float32)
    h = jnp.maximum(h + b1_ref[...], 0.0)
    o = jnp.dot(h, w2_ref[...], preferred_element_type=jnp.float32)
    out_ref[...] = o + b2_ref[...]


def _mlp(rep_sum, lenf, W1, b1, W2, b2):
    H_PAD = 128
    O_PAD = 128
    W1p = jnp.zeros((D, H_PAD), jnp.float32).at[:, :HIDDEN].set(W1)
    b1p = jnp.zeros((1, H_PAD), jnp.float32).at[:, :HIDDEN].set(b1)
    W2p = jnp.zeros((H_PAD, O_PAD), jnp.float32).at[:HIDDEN, :OUT].set(W2)
    b2p = jnp.zeros((1, O_PAD), jnp.float32).at[:, :OUT].set(b2)
    BLK = 2048
    grid = (B // BLK,)
    out = pl.pallas_call(
        _mlp_body,
        grid=grid,
        in_specs=[
            pl.BlockSpec((BLK, D), lambda i: (i, 0)),
            pl.BlockSpec((BLK, 1), lambda i: (i, 0)),
            pl.BlockSpec((D, H_PAD), lambda i: (0, 0)),
            pl.BlockSpec((1, H_PAD), lambda i: (0, 0)),
            pl.BlockSpec((H_PAD, O_PAD), lambda i: (0, 0)),
            pl.BlockSpec((1, O_PAD), lambda i: (0, 0)),
        ],
        out_specs=pl.BlockSpec((BLK, O_PAD), lambda i: (i, 0)),
        out_shape=jax.ShapeDtypeStruct((B, O_PAD), jnp.float32),
    )(rep_sum, lenf, W1p, b1p, W2p, b2p)
    return out[:, :OUT]


def kernel(x, lengths, table, W1, b1, W2, b2):
    rep_sum = _pooled_sum(x, table)
    lenf = lengths.astype(jnp.float32).reshape(B, 1)
    return _mlp(rep_sum, lenf, W1, b1, W2, b2)
